# baseline (speedup 1.0000x reference)
import jax
import jax.numpy as jnp
from jax.experimental import pallas as pl
from jax.experimental.pallas import tpu as pltpu

_N = 10000
_D = 128
_BM = 200
_NB = _N // _BM
_STEPS = 2 * _NB
_NBUF = 5


def _seq_row(seq):
    return jnp.where(seq < _NB, seq, seq - _NB)


def _start_fetch(adj_ref, abuf, sems, seq):
    slot = jax.lax.rem(seq, _NBUF)
    row = _seq_row(seq) * _BM
    pltpu.make_async_copy(
        adj_ref.at[pl.ds(row, _BM), :], abuf.at[slot], sems.at[slot]
    ).start()


def _wait_fetch(adj_ref, abuf, sems, seq):
    slot = jax.lax.rem(seq, _NBUF)
    row = _seq_row(seq) * _BM
    pltpu.make_async_copy(
        adj_ref.at[pl.ds(row, _BM), :], abuf.at[slot], sems.at[slot]
    ).wait()
    return slot


def _fused_kernel(x_ref, adj_ref, w1_ref, b1_ref, w2_ref, b2_ref,
                  w3_ref, b3_ref, w4_ref, b4_ref,
                  emb_ref, z_ref, g1_ref, g2_ref, abuf, sems):
    i = pl.program_id(0)

    @pl.when(i == 0)
    def _g1_phase():
        for t in range(_NBUF - 1):
            _start_fetch(adj_ref, abuf, sems, t)
        acc = jnp.dot(x_ref[...], w1_ref[...],
                      preferred_element_type=jnp.float32) + b1_ref[...]
        g1_ref[...] = acc.astype(jnp.bfloat16)

    @pl.when(i >= 1)
    def _stream_phase():
        s = i - 1

        @pl.when(s + _NBUF - 1 < _STEPS)
        def _():
            _start_fetch(adj_ref, abuf, sems, s + _NBUF - 1)

        slot = _wait_fetch(adj_ref, abuf, sems, s)
        a = abuf[slot].astype(jnp.bfloat16)

        @pl.when(s < _NB)
        def _pass1():
            h = jnp.dot(a, g1_ref[...], preferred_element_type=jnp.float32)
            h = jnp.maximum(h, 0.0)
            g2 = jnp.dot(h, w2_ref[...],
                         preferred_element_type=jnp.float32) + b2_ref[...]
            g2_ref[pl.ds(s * _BM, _BM), :] = g2.astype(jnp.bfloat16)

        @pl.when(s >= _NB)
        def _pass2():
            emb = jnp.dot(a, g2_ref[...], preferred_element_type=jnp.float32)
            emb_ref[...] = emb
            t = jnp.dot(emb, w3_ref[...],
                        preferred_element_type=jnp.float32) + b3_ref[...]
            t = jnp.maximum(t, 0.0)
            z_ref[...] = jnp.dot(t, w4_ref[...],
                                 preferred_element_type=jnp.float32) + b4_ref[...]


def _out_map(i):
    return (jnp.clip(i - 1 - _NB, 0, _NB - 1), 0)


def _const_map(i):
    return (0, 0)


def kernel(x, Adj_, W1, b1, W2, b2, W3, b3, W4, b4):
    full = lambda r, c: pl.BlockSpec((r, c), _const_map)
    emb, z = pl.pallas_call(
        _fused_kernel,
        grid=(1 + _STEPS,),
        in_specs=[
            full(_N, _D),
            pl.BlockSpec(memory_space=pltpu.MemorySpace.HBM),
            full(_D, _D), full(1, _D),
            full(_D, _D), full(1, _D),
            full(_D, _D), full(1, _D),
            full(_D, _D), full(1, _D),
        ],
        out_specs=[
            pl.BlockSpec((_BM, _D), _out_map),
            pl.BlockSpec((_BM, _D), _out_map),
        ],
        out_shape=[
            jax.ShapeDtypeStruct((_N, _D), jnp.float32),
            jax.ShapeDtypeStruct((_N, _D), jnp.float32),
        ],
        scratch_shapes=[
            pltpu.VMEM((_N, _D), jnp.bfloat16),
            pltpu.VMEM((_N, _D), jnp.bfloat16),
            pltpu.VMEM((_NBUF, _BM, _N), jnp.float32),
            pltpu.SemaphoreType.DMA((_NBUF,)),
        ],
    )(x, Adj_, W1, b1.reshape(1, _D), W2, b2.reshape(1, _D),
      W3, b3.reshape(1, _D), W4, b4.reshape(1, _D))
    return (z, emb)
